# baseline (speedup 1.0000x reference)
import functools

import jax
import jax.numpy as jnp
from jax import lax
from jax.experimental import pallas as pl
from jax.experimental.pallas import tpu as pltpu
from jax.experimental.pallas import tpu_sc as plsc

_INTERPRET = False
_PREC = jax.lax.Precision.DEFAULT


def _three_nn_kernel(q_ref, p_ref, ind_ref, w_ref, *, n_keys, n_tiles,
                     b_base=0):
    q = q_ref[0]
    p = p_ref[0]
    pp = jnp.sum(p * p, axis=1, keepdims=True)
    qp = jax.lax.dot_general(p, q, (((1,), (0,)), ((), ())),
                             precision=jax.lax.Precision.DEFAULT)
    s = pp - 2.0 * qp
    qq = jnp.sum(q * q, axis=0, keepdims=True)
    iota = jax.lax.broadcasted_iota(jnp.int32, s.shape, 0)
    sentinel = n_keys
    mins, args = [], []
    cur = s
    for k in range(3):
        m = jnp.min(cur, axis=0, keepdims=True)
        a = jnp.min(jnp.where(cur == m, iota, sentinel),
                    axis=0, keepdims=True)
        mins.append(m)
        args.append(a)
        if k < 2:
            cur = jnp.where(iota == a, jnp.inf, cur)
    sqs = [jnp.maximum(m + qq, 0.0) for m in mins]
    dists = [jnp.where(d < 1e-10, 1e-10, d) for d in sqs]
    invs = [1.0 / (d + 1e-8) for d in dists]
    norm = invs[0] + invs[1] + invs[2]
    boff = (b_base + pl.program_id(0) // n_tiles) * n_keys
    ind_ref[...] = jnp.concatenate(args, axis=0) + boff
    w_ref[...] = jnp.concatenate([iv / norm for iv in invs], axis=0)


def _sc_interp(table, idx, wts, *, q_total, d_feat):
    info = plsc.get_sparse_core_info()
    nw = info.num_cores * info.num_subcores
    q_per_w = q_total // nw
    G = 32
    n_chunks = q_per_w // G
    nl = 16
    mesh = plsc.VectorSubcoreMesh(core_axis_name="c", subcore_axis_name="s")

    idx4 = idx.reshape(3, nw, n_chunks, G)
    wts4 = wts.reshape(3, nw, 1, q_per_w)

    @functools.partial(
        pl.kernel, mesh=mesh,
        out_type=jax.ShapeDtypeStruct((q_total, d_feat), jnp.float32),
        scratch_types=[pltpu.VMEM((3, n_chunks, G), jnp.int32),
                       pltpu.VMEM((3, 1, q_per_w + nl), jnp.float32),
                       pltpu.VMEM((G, d_feat), jnp.float32),
                       pltpu.VMEM((G, d_feat), jnp.float32),
                       pltpu.VMEM((G, d_feat), jnp.float32),
                       pltpu.VMEM((G, d_feat), jnp.float32),
                       pltpu.VMEM((G, d_feat), jnp.float32),
                       pltpu.VMEM((G, d_feat), jnp.float32),
                       pltpu.VMEM((G, d_feat), jnp.float32),
                       pltpu.VMEM((G, d_feat), jnp.float32),
                       pltpu.SemaphoreType.DMA,
                       pltpu.SemaphoreType.DMA,
                       pltpu.SemaphoreType.DMA,
                       pltpu.SemaphoreType.DMA],
    )
    def gather_kernel(table_hbm, idx_hbm, w_hbm, out_hbm,
                      idx_v, w_v, ra0, ra1, ra2, rb0, rb1, rb2,
                      ova, ovb, gsa, gsb, wsa, wsb):
        wid = lax.axis_index("s") * info.num_cores + lax.axis_index("c")
        wbase = wid * q_per_w
        rbufs = ((ra0, ra1, ra2), (rb0, rb1, rb2))
        ovs = (ova, ovb)
        gsems = (gsa, gsb)
        wsems = (wsa, wsb)

        for k in range(3):
            pltpu.sync_copy(idx_hbm.at[k, wid], idx_v.at[k])
            pltpu.sync_copy(w_hbm.at[k, wid],
                            w_v.at[k, pl.ds(0, 1), pl.ds(0, q_per_w)])

        def fire(c, side):
            for k in range(3):
                pltpu.async_copy(table_hbm.at[idx_v.at[k, c]],
                                 rbufs[side][k], gsems[side])

        def drain_gather(side):
            for k in range(3):
                pltpu.make_async_copy(table_hbm.at[pl.ds(0, G)],
                                      rbufs[side][k], gsems[side]).wait()

        def compute(c, side):
            r0, r1, r2 = rbufs[side]
            ov = ovs[side]

            def q_body(g, qcarry):
                qi = c * G + g
                wv0 = lax.broadcast(w_v[0, 0, pl.ds(qi, nl)][0], (nl,))
                wv1 = lax.broadcast(w_v[1, 0, pl.ds(qi, nl)][0], (nl,))
                wv2 = lax.broadcast(w_v[2, 0, pl.ds(qi, nl)][0], (nl,))
                for cc in range(d_feat // nl):
                    sl = pl.ds(cc * nl, nl)
                    ov[g, sl] = (wv0 * r0[g, sl] + wv1 * r1[g, sl]
                                 + wv2 * r2[g, sl])
                return qcarry

            lax.fori_loop(0, G, q_body, 0)

        def fire_wb(c, side):
            pltpu.async_copy(ovs[side], out_hbm.at[pl.ds(wbase + c * G, G)],
                             wsems[side])

        def drain_wb(side):
            pltpu.make_async_copy(table_hbm.at[pl.ds(0, G)], ovs[side],
                                  wsems[side]).wait()

        fire(0, 0)
        fire(1, 1)

        def pair_body(i, carry):
            c0 = 2 * i
            for side in range(2):
                c = c0 + side
                drain_gather(side)

                @pl.when(i > 0)
                def _():
                    drain_wb(side)

                compute(c, side)
                fire_wb(c, side)

                @pl.when(c + 2 < n_chunks)
                def _():
                    fire(c + 2, side)

            return carry

        lax.fori_loop(0, n_chunks // 2, pair_body, 0)
        drain_wb(0)
        drain_wb(1)

    return gather_kernel(table, idx4, wts4)


def _transpose_kernel(x_ref, o_ref):
    o_ref[0] = x_ref[0].T


def _conv1_kernel(f_ref, skip_ref, W1_ref, y1_ref, st_ref, *, cprev):
    b = pl.program_id(0)
    t = pl.program_id(1)

    @pl.when(jnp.logical_and(b == 0, t == 0))
    def _init():
        st_ref[...] = jnp.zeros_like(st_ref)

    x = f_ref[0]
    skipb = skip_ref[0]
    W1 = W1_ref[...]
    y1 = (jax.lax.dot_general(x, W1[:, :cprev], (((1,), (1,)), ((), ())),
                              precision=_PREC)
          + jax.lax.dot_general(skipb, W1[:, cprev:], (((0,), (1,)), ((), ())),
                                precision=_PREC))
    y1_ref[0] = y1
    s1 = jnp.sum(y1, axis=0, keepdims=True)
    s2 = jnp.sum(y1 * y1, axis=0, keepdims=True)
    st_ref[...] += jnp.concatenate([s1, s2], axis=0)


def _bn_conv2_kernel(y1_ref, st_ref, g_ref, b_ref, W2_ref,
                     y2_ref, st2_ref, *, count):
    b = pl.program_id(0)
    t = pl.program_id(1)

    @pl.when(jnp.logical_and(b == 0, t == 0))
    def _init():
        st2_ref[...] = jnp.zeros_like(st2_ref)

    st = st_ref[...]
    inv_cnt = 1.0 / count
    mean = st[0:1, :] * inv_cnt
    var = st[1:2, :] * inv_cnt - mean * mean
    inv = jax.lax.rsqrt(var + 1e-3)
    z = (y1_ref[0] - mean) * inv * g_ref[...] + b_ref[...]
    z = jnp.where(z >= 0, z, 0.01 * z)
    y2 = jax.lax.dot_general(z, W2_ref[...], (((1,), (1,)), ((), ())),
                             precision=_PREC)
    y2_ref[0] = y2
    s1 = jnp.sum(y2, axis=0, keepdims=True)
    s2 = jnp.sum(y2 * y2, axis=0, keepdims=True)
    st2_ref[...] += jnp.concatenate([s1, s2], axis=0)


def _bn_out_kernel(y2_ref, st_ref, g_ref, b_ref, out_ref, *, count):
    st = st_ref[...]
    inv_cnt = 1.0 / count
    mean = st[0:1, :] * inv_cnt
    var = st[1:2, :] * inv_cnt - mean * mean
    inv = jax.lax.rsqrt(var + 1e-3)
    z = (y2_ref[0] - mean) * inv * g_ref[...] + b_ref[...]
    z = jnp.where(z >= 0, z, 0.01 * z)
    out_ref[0] = z.T


def kernel(xyz, skip, xyz_prev, feat_prev, W1, g1, b1, W2, g2, b2):
    B, _, N0 = xyz.shape
    N = xyz_prev.shape[2]
    Cprev = feat_prev.shape[1]
    Cskip = skip.shape[1]
    C1 = W1.shape[0]
    C2 = W2.shape[0]
    TQ = 1024
    nt = N0 // TQ
    TC = 1024
    ntc = N0 // TC
    Q = B * N0
    count = float(Q)

    p_t = jnp.transpose(xyz_prev, (0, 2, 1))

    TT = 512
    table = pl.pallas_call(
        _transpose_kernel,
        grid=(B, N // TT),
        in_specs=[pl.BlockSpec((1, Cprev, TT), lambda b, t: (b, 0, t))],
        out_specs=pl.BlockSpec((1, TT, Cprev), lambda b, t: (b, t, 0)),
        out_shape=jax.ShapeDtypeStruct((B, N, Cprev), jnp.float32),
        interpret=_INTERPRET,
    )(feat_prev).reshape(B * N, Cprev)

    idx_flat, w_flat = pl.pallas_call(
        functools.partial(_three_nn_kernel, n_keys=N, n_tiles=nt),
        grid=(B * nt,),
        in_specs=[pl.BlockSpec((1, 3, TQ), lambda i: (i // nt, 0, i % nt)),
                  pl.BlockSpec((1, N, 3), lambda i: (i // nt, 0, 0))],
        out_specs=[pl.BlockSpec((3, TQ), lambda i: (0, i)),
                   pl.BlockSpec((3, TQ), lambda i: (0, i))],
        out_shape=[jax.ShapeDtypeStruct((3, Q), jnp.int32),
                   jax.ShapeDtypeStruct((3, Q), jnp.float32)],
        interpret=_INTERPRET,
    )(xyz, p_t)

    feats = _sc_interp(table, idx_flat, w_flat, q_total=Q, d_feat=Cprev)
    feats = feats.reshape(B, N0, Cprev)

    y1, st1 = pl.pallas_call(
        functools.partial(_conv1_kernel, cprev=Cprev),
        grid=(B, ntc),
        in_specs=[pl.BlockSpec((1, TC, Cprev), lambda b, t: (b, t, 0)),
                  pl.BlockSpec((1, Cskip, TC), lambda b, t: (b, 0, t)),
                  pl.BlockSpec((C1, Cprev + Cskip), lambda b, t: (0, 0))],
        out_specs=[pl.BlockSpec((1, TC, C1), lambda b, t: (b, t, 0)),
                   pl.BlockSpec((2, C1), lambda b, t: (0, 0))],
        out_shape=[jax.ShapeDtypeStruct((B, N0, C1), jnp.float32),
                   jax.ShapeDtypeStruct((2, C1), jnp.float32)],
        interpret=_INTERPRET,
    )(feats, skip, W1)

    y2, st2 = pl.pallas_call(
        functools.partial(_bn_conv2_kernel, count=count),
        grid=(B, ntc),
        in_specs=[pl.BlockSpec((1, TC, C1), lambda b, t: (b, t, 0)),
                  pl.BlockSpec((2, C1), lambda b, t: (0, 0)),
                  pl.BlockSpec((1, C1), lambda b, t: (0, 0)),
                  pl.BlockSpec((1, C1), lambda b, t: (0, 0)),
                  pl.BlockSpec((C2, C1), lambda b, t: (0, 0))],
        out_specs=[pl.BlockSpec((1, TC, C2), lambda b, t: (b, t, 0)),
                   pl.BlockSpec((2, C2), lambda b, t: (0, 0))],
        out_shape=[jax.ShapeDtypeStruct((B, N0, C2), jnp.float32),
                   jax.ShapeDtypeStruct((2, C2), jnp.float32)],
        interpret=_INTERPRET,
    )(y1, st1, g1.reshape(1, -1), b1.reshape(1, -1), W2)

    y = pl.pallas_call(
        functools.partial(_bn_out_kernel, count=count),
        grid=(B, ntc),
        in_specs=[pl.BlockSpec((1, TC, C2), lambda b, t: (b, t, 0)),
                  pl.BlockSpec((2, C2), lambda b, t: (0, 0)),
                  pl.BlockSpec((1, C2), lambda b, t: (0, 0)),
                  pl.BlockSpec((1, C2), lambda b, t: (0, 0))],
        out_specs=pl.BlockSpec((1, C2, TC), lambda b, t: (b, 0, t)),
        out_shape=jax.ShapeDtypeStruct((B, C2, N0), jnp.float32),
        interpret=_INTERPRET,
    )(y2, st2, g2.reshape(1, -1), b2.reshape(1, -1))

    return (xyz, y)


# speedup vs baseline: 1.0692x; 1.0692x over previous
import functools

import jax
import jax.numpy as jnp
from jax import lax
from jax.experimental import pallas as pl
from jax.experimental.pallas import tpu as pltpu
from jax.experimental.pallas import tpu_sc as plsc

_INTERPRET = False
_PREC = jax.lax.Precision.DEFAULT


def _three_nn_kernel(q_ref, p_ref, ind_ref, w_ref, *, n_keys, n_tiles,
                     b_base=0):
    q = q_ref[0]
    p = p_ref[0]
    pp = jnp.sum(p * p, axis=1, keepdims=True)
    qp = jax.lax.dot_general(p, q, (((1,), (0,)), ((), ())),
                             precision=jax.lax.Precision.DEFAULT)
    s = pp - 2.0 * qp
    qq = jnp.sum(q * q, axis=0, keepdims=True)
    iota = jax.lax.broadcasted_iota(jnp.int32, s.shape, 0)
    sentinel = n_keys
    mins, args = [], []
    cur = s
    for k in range(3):
        m = jnp.min(cur, axis=0, keepdims=True)
        a = jnp.min(jnp.where(cur == m, iota, sentinel),
                    axis=0, keepdims=True)
        mins.append(m)
        args.append(a)
        if k < 2:
            cur = jnp.where(cur == m, jnp.inf, cur)
    sqs = [jnp.maximum(m + qq, 0.0) for m in mins]
    dists = [jnp.where(d < 1e-10, 1e-10, d) for d in sqs]
    invs = [1.0 / (d + 1e-8) for d in dists]
    norm = invs[0] + invs[1] + invs[2]
    boff = (b_base + pl.program_id(0) // n_tiles) * n_keys
    ind_ref[...] = jnp.concatenate(args, axis=0) + boff
    w_ref[...] = jnp.concatenate([iv / norm for iv in invs], axis=0)


def _sc_interp(table, idx, wts, *, q_total, d_feat):
    info = plsc.get_sparse_core_info()
    nw = info.num_cores * info.num_subcores
    q_per_w = q_total // nw
    G = 32
    n_chunks = q_per_w // G
    nl = 16
    mesh = plsc.VectorSubcoreMesh(core_axis_name="c", subcore_axis_name="s")

    idx4 = idx.reshape(3, nw, n_chunks, G)
    wts4 = wts.reshape(3, nw, 1, q_per_w)

    @functools.partial(
        pl.kernel, mesh=mesh,
        out_type=jax.ShapeDtypeStruct((q_total, d_feat), jnp.float32),
        scratch_types=[pltpu.VMEM((3, n_chunks, G), jnp.int32),
                       pltpu.VMEM((3, 1, q_per_w + nl), jnp.float32),
                       pltpu.VMEM((G, d_feat), jnp.float32),
                       pltpu.VMEM((G, d_feat), jnp.float32),
                       pltpu.VMEM((G, d_feat), jnp.float32),
                       pltpu.VMEM((G, d_feat), jnp.float32),
                       pltpu.VMEM((G, d_feat), jnp.float32),
                       pltpu.VMEM((G, d_feat), jnp.float32),
                       pltpu.VMEM((G, d_feat), jnp.float32),
                       pltpu.VMEM((G, d_feat), jnp.float32),
                       pltpu.SemaphoreType.DMA,
                       pltpu.SemaphoreType.DMA,
                       pltpu.SemaphoreType.DMA,
                       pltpu.SemaphoreType.DMA],
    )
    def gather_kernel(table_hbm, idx_hbm, w_hbm, out_hbm,
                      idx_v, w_v, ra0, ra1, ra2, rb0, rb1, rb2,
                      ova, ovb, gsa, gsb, wsa, wsb):
        wid = lax.axis_index("s") * info.num_cores + lax.axis_index("c")
        wbase = wid * q_per_w
        rbufs = ((ra0, ra1, ra2), (rb0, rb1, rb2))
        ovs = (ova, ovb)
        gsems = (gsa, gsb)
        wsems = (wsa, wsb)

        for k in range(3):
            pltpu.sync_copy(idx_hbm.at[k, wid], idx_v.at[k])
            pltpu.sync_copy(w_hbm.at[k, wid],
                            w_v.at[k, pl.ds(0, 1), pl.ds(0, q_per_w)])

        def fire(c, side):
            for k in range(3):
                pltpu.async_copy(table_hbm.at[idx_v.at[k, c]],
                                 rbufs[side][k], gsems[side])

        def drain_gather(side):
            for k in range(3):
                pltpu.make_async_copy(table_hbm.at[pl.ds(0, G)],
                                      rbufs[side][k], gsems[side]).wait()

        def compute(c, side):
            r0, r1, r2 = rbufs[side]
            ov = ovs[side]

            def q_body(g, qcarry):
                qi = c * G + g
                wv0 = lax.broadcast(w_v[0, 0, pl.ds(qi, nl)][0], (nl,))
                wv1 = lax.broadcast(w_v[1, 0, pl.ds(qi, nl)][0], (nl,))
                wv2 = lax.broadcast(w_v[2, 0, pl.ds(qi, nl)][0], (nl,))
                for cc in range(d_feat // nl):
                    sl = pl.ds(cc * nl, nl)
                    ov[g, sl] = (wv0 * r0[g, sl] + wv1 * r1[g, sl]
                                 + wv2 * r2[g, sl])
                return qcarry

            lax.fori_loop(0, G, q_body, 0)

        def fire_wb(c, side):
            pltpu.async_copy(ovs[side], out_hbm.at[pl.ds(wbase + c * G, G)],
                             wsems[side])

        def drain_wb(side):
            pltpu.make_async_copy(table_hbm.at[pl.ds(0, G)], ovs[side],
                                  wsems[side]).wait()

        fire(0, 0)
        fire(1, 1)

        def pair_body(i, carry):
            c0 = 2 * i
            for side in range(2):
                c = c0 + side
                drain_gather(side)

                @pl.when(i > 0)
                def _():
                    drain_wb(side)

                compute(c, side)
                fire_wb(c, side)

                @pl.when(c + 2 < n_chunks)
                def _():
                    fire(c + 2, side)

            return carry

        lax.fori_loop(0, n_chunks // 2, pair_body, 0)
        drain_wb(0)
        drain_wb(1)

    return gather_kernel(table, idx4, wts4)


def _conv1_kernel(f_ref, skip_ref, W1_ref, y1_ref, st_ref, *, cprev):
    b = pl.program_id(0)
    t = pl.program_id(1)

    @pl.when(jnp.logical_and(b == 0, t == 0))
    def _init():
        st_ref[...] = jnp.zeros_like(st_ref)

    x = f_ref[0]
    skipb = skip_ref[0]
    W1 = W1_ref[...]
    y1 = (jax.lax.dot_general(x, W1[:, :cprev], (((1,), (1,)), ((), ())),
                              precision=_PREC)
          + jax.lax.dot_general(skipb, W1[:, cprev:], (((0,), (1,)), ((), ())),
                                precision=_PREC))
    y1_ref[0] = y1
    s1 = jnp.sum(y1, axis=0, keepdims=True)
    s2 = jnp.sum(y1 * y1, axis=0, keepdims=True)
    st_ref[...] += jnp.concatenate([s1, s2], axis=0)


def _bn_conv2_kernel(y1_ref, st_ref, g_ref, b_ref, W2_ref,
                     y2_ref, st2_ref, *, count):
    b = pl.program_id(0)
    t = pl.program_id(1)

    @pl.when(jnp.logical_and(b == 0, t == 0))
    def _init():
        st2_ref[...] = jnp.zeros_like(st2_ref)

    st = st_ref[...]
    inv_cnt = 1.0 / count
    mean = st[0:1, :] * inv_cnt
    var = st[1:2, :] * inv_cnt - mean * mean
    inv = jax.lax.rsqrt(var + 1e-3)
    z = (y1_ref[0] - mean) * inv * g_ref[...] + b_ref[...]
    z = jnp.where(z >= 0, z, 0.01 * z)
    y2 = jax.lax.dot_general(z, W2_ref[...], (((1,), (1,)), ((), ())),
                             precision=_PREC)
    y2_ref[0] = y2
    s1 = jnp.sum(y2, axis=0, keepdims=True)
    s2 = jnp.sum(y2 * y2, axis=0, keepdims=True)
    st2_ref[...] += jnp.concatenate([s1, s2], axis=0)


def _bn_out_kernel(y2_ref, st_ref, g_ref, b_ref, out_ref, *, count):
    st = st_ref[...]
    inv_cnt = 1.0 / count
    mean = st[0:1, :] * inv_cnt
    var = st[1:2, :] * inv_cnt - mean * mean
    inv = jax.lax.rsqrt(var + 1e-3)
    z = (y2_ref[0] - mean) * inv * g_ref[...] + b_ref[...]
    z = jnp.where(z >= 0, z, 0.01 * z)
    out_ref[0] = z.T


def kernel(xyz, skip, xyz_prev, feat_prev, W1, g1, b1, W2, g2, b2):
    B, _, N0 = xyz.shape
    N = xyz_prev.shape[2]
    Cprev = feat_prev.shape[1]
    Cskip = skip.shape[1]
    C1 = W1.shape[0]
    C2 = W2.shape[0]
    TQ = 1024
    nt = N0 // TQ
    TC = 1024
    ntc = N0 // TC
    Q = B * N0
    count = float(Q)

    p_t = jnp.transpose(xyz_prev, (0, 2, 1))

    table = jnp.transpose(feat_prev, (0, 2, 1)).reshape(B * N, Cprev)

    idx_flat, w_flat = pl.pallas_call(
        functools.partial(_three_nn_kernel, n_keys=N, n_tiles=nt),
        grid=(B * nt,),
        in_specs=[pl.BlockSpec((1, 3, TQ), lambda i: (i // nt, 0, i % nt)),
                  pl.BlockSpec((1, N, 3), lambda i: (i // nt, 0, 0))],
        out_specs=[pl.BlockSpec((3, TQ), lambda i: (0, i)),
                   pl.BlockSpec((3, TQ), lambda i: (0, i))],
        out_shape=[jax.ShapeDtypeStruct((3, Q), jnp.int32),
                   jax.ShapeDtypeStruct((3, Q), jnp.float32)],
        interpret=_INTERPRET,
    )(xyz, p_t)

    feats = _sc_interp(table, idx_flat, w_flat, q_total=Q, d_feat=Cprev)
    feats = feats.reshape(B, N0, Cprev)

    y1, st1 = pl.pallas_call(
        functools.partial(_conv1_kernel, cprev=Cprev),
        grid=(B, ntc),
        in_specs=[pl.BlockSpec((1, TC, Cprev), lambda b, t: (b, t, 0)),
                  pl.BlockSpec((1, Cskip, TC), lambda b, t: (b, 0, t)),
                  pl.BlockSpec((C1, Cprev + Cskip), lambda b, t: (0, 0))],
        out_specs=[pl.BlockSpec((1, TC, C1), lambda b, t: (b, t, 0)),
                   pl.BlockSpec((2, C1), lambda b, t: (0, 0))],
        out_shape=[jax.ShapeDtypeStruct((B, N0, C1), jnp.float32),
                   jax.ShapeDtypeStruct((2, C1), jnp.float32)],
        interpret=_INTERPRET,
    )(feats, skip, W1)

    y2, st2 = pl.pallas_call(
        functools.partial(_bn_conv2_kernel, count=count),
        grid=(B, ntc),
        in_specs=[pl.BlockSpec((1, TC, C1), lambda b, t: (b, t, 0)),
                  pl.BlockSpec((2, C1), lambda b, t: (0, 0)),
                  pl.BlockSpec((1, C1), lambda b, t: (0, 0)),
                  pl.BlockSpec((1, C1), lambda b, t: (0, 0)),
                  pl.BlockSpec((C2, C1), lambda b, t: (0, 0))],
        out_specs=[pl.BlockSpec((1, TC, C2), lambda b, t: (b, t, 0)),
                   pl.BlockSpec((2, C2), lambda b, t: (0, 0))],
        out_shape=[jax.ShapeDtypeStruct((B, N0, C2), jnp.float32),
                   jax.ShapeDtypeStruct((2, C2), jnp.float32)],
        interpret=_INTERPRET,
    )(y1, st1, g1.reshape(1, -1), b1.reshape(1, -1), W2)

    y = pl.pallas_call(
        functools.partial(_bn_out_kernel, count=count),
        grid=(B, ntc),
        in_specs=[pl.BlockSpec((1, TC, C2), lambda b, t: (b, t, 0)),
                  pl.BlockSpec((2, C2), lambda b, t: (0, 0)),
                  pl.BlockSpec((1, C2), lambda b, t: (0, 0)),
                  pl.BlockSpec((1, C2), lambda b, t: (0, 0))],
        out_specs=pl.BlockSpec((1, C2, TC), lambda b, t: (b, 0, t)),
        out_shape=jax.ShapeDtypeStruct((B, C2, N0), jnp.float32),
        interpret=_INTERPRET,
    )(y2, st2, g2.reshape(1, -1), b2.reshape(1, -1))

    return (xyz, y)


# speedup vs baseline: 1.0694x; 1.0002x over previous
import functools

import jax
import jax.numpy as jnp
from jax import lax
from jax.experimental import pallas as pl
from jax.experimental.pallas import tpu as pltpu
from jax.experimental.pallas import tpu_sc as plsc

_PREC = jax.lax.Precision.DEFAULT


def _three_nn_kernel(q_ref, p_ref, ind_ref, w_ref, *, n_keys, n_tiles,
                     b_base=0):
    q = q_ref[0]
    p = p_ref[0]
    pp = jnp.sum(p * p, axis=1, keepdims=True)
    qp = jax.lax.dot_general(p, q, (((1,), (0,)), ((), ())),
                             precision=jax.lax.Precision.DEFAULT)
    s = pp - 2.0 * qp
    qq = jnp.sum(q * q, axis=0, keepdims=True)
    iota = jax.lax.broadcasted_iota(jnp.int32, s.shape, 0)
    sentinel = n_keys
    mins, args = [], []
    cur = s
    for k in range(3):
        m = jnp.min(cur, axis=0, keepdims=True)
        a = jnp.min(jnp.where(cur == m, iota, sentinel),
                    axis=0, keepdims=True)
        mins.append(m)
        args.append(a)
        if k < 2:
            cur = jnp.where(cur == m, jnp.inf, cur)
    sqs = [jnp.maximum(m + qq, 0.0) for m in mins]
    dists = [jnp.where(d < 1e-10, 1e-10, d) for d in sqs]
    invs = [1.0 / (d + 1e-8) for d in dists]
    norm = invs[0] + invs[1] + invs[2]
    boff = (b_base + pl.program_id(0) // n_tiles) * n_keys
    ind_ref[...] = jnp.concatenate(args, axis=0) + boff
    w_ref[...] = jnp.concatenate([iv / norm for iv in invs], axis=0)


def _sc_interp(table, idx, wts, *, q_total, d_feat):
    info = plsc.get_sparse_core_info()
    nw = info.num_cores * info.num_subcores
    q_per_w = q_total // nw
    G = 32
    n_chunks = q_per_w // G
    nl = 16
    mesh = plsc.VectorSubcoreMesh(core_axis_name="c", subcore_axis_name="s")

    idx4 = idx.reshape(3, nw, n_chunks, G)
    wts4 = wts.reshape(3, nw, 1, q_per_w)

    @functools.partial(
        pl.kernel, mesh=mesh,
        out_type=jax.ShapeDtypeStruct((q_total, d_feat), jnp.float32),
        scratch_types=[pltpu.VMEM((3, n_chunks, G), jnp.int32),
                       pltpu.VMEM((3, 1, q_per_w + nl), jnp.float32),
                       pltpu.VMEM((G, d_feat), jnp.float32),
                       pltpu.VMEM((G, d_feat), jnp.float32),
                       pltpu.VMEM((G, d_feat), jnp.float32),
                       pltpu.VMEM((G, d_feat), jnp.float32),
                       pltpu.VMEM((G, d_feat), jnp.float32),
                       pltpu.VMEM((G, d_feat), jnp.float32),
                       pltpu.VMEM((G, d_feat), jnp.float32),
                       pltpu.VMEM((G, d_feat), jnp.float32),
                       pltpu.SemaphoreType.DMA,
                       pltpu.SemaphoreType.DMA,
                       pltpu.SemaphoreType.DMA,
                       pltpu.SemaphoreType.DMA],
    )
    def gather_kernel(table_hbm, idx_hbm, w_hbm, out_hbm,
                      idx_v, w_v, ra0, ra1, ra2, rb0, rb1, rb2,
                      ova, ovb, gsa, gsb, wsa, wsb):
        wid = lax.axis_index("s") * info.num_cores + lax.axis_index("c")
        wbase = wid * q_per_w
        rbufs = ((ra0, ra1, ra2), (rb0, rb1, rb2))
        ovs = (ova, ovb)
        gsems = (gsa, gsb)
        wsems = (wsa, wsb)

        for k in range(3):
            pltpu.sync_copy(idx_hbm.at[k, wid], idx_v.at[k])
            pltpu.sync_copy(w_hbm.at[k, wid],
                            w_v.at[k, pl.ds(0, 1), pl.ds(0, q_per_w)])

        def fire(c, side):
            for k in range(3):
                pltpu.async_copy(table_hbm.at[idx_v.at[k, c]],
                                 rbufs[side][k], gsems[side])

        def drain_gather(side):
            for k in range(3):
                pltpu.make_async_copy(table_hbm.at[pl.ds(0, G)],
                                      rbufs[side][k], gsems[side]).wait()

        def compute(c, side):
            r0, r1, r2 = rbufs[side]
            ov = ovs[side]

            def q_body(g, qcarry):
                qi = c * G + g
                wv0 = lax.broadcast(w_v[0, 0, pl.ds(qi, nl)][0], (nl,))
                wv1 = lax.broadcast(w_v[1, 0, pl.ds(qi, nl)][0], (nl,))
                wv2 = lax.broadcast(w_v[2, 0, pl.ds(qi, nl)][0], (nl,))
                for cc in range(d_feat // nl):
                    sl = pl.ds(cc * nl, nl)
                    ov[g, sl] = (wv0 * r0[g, sl] + wv1 * r1[g, sl]
                                 + wv2 * r2[g, sl])
                return qcarry

            lax.fori_loop(0, G, q_body, 0)

        def fire_wb(c, side):
            pltpu.async_copy(ovs[side], out_hbm.at[pl.ds(wbase + c * G, G)],
                             wsems[side])

        def drain_wb(side):
            pltpu.make_async_copy(table_hbm.at[pl.ds(0, G)], ovs[side],
                                  wsems[side]).wait()

        fire(0, 0)
        fire(1, 1)

        def pair_body(i, carry):
            c0 = 2 * i
            for side in range(2):
                c = c0 + side
                drain_gather(side)

                @pl.when(i > 0)
                def _():
                    drain_wb(side)

                compute(c, side)
                fire_wb(c, side)

                @pl.when(c + 2 < n_chunks)
                def _():
                    fire(c + 2, side)

            return carry

        lax.fori_loop(0, n_chunks // 2, pair_body, 0)
        drain_wb(0)
        drain_wb(1)

    return gather_kernel(table, idx4, wts4)


def _conv1_kernel(f_ref, skip_ref, W1_ref, y1_ref, st_ref, *, cprev):
    b = pl.program_id(0)
    t = pl.program_id(1)

    @pl.when(jnp.logical_and(b == 0, t == 0))
    def _init():
        st_ref[...] = jnp.zeros_like(st_ref)

    x = f_ref[0]
    skipb = skip_ref[0]
    W1 = W1_ref[...]
    y1 = (jax.lax.dot_general(x, W1[:, :cprev], (((1,), (1,)), ((), ())),
                              precision=_PREC)
          + jax.lax.dot_general(skipb, W1[:, cprev:], (((0,), (1,)), ((), ())),
                                precision=_PREC))
    y1_ref[0] = y1
    s1 = jnp.sum(y1, axis=0, keepdims=True)
    s2 = jnp.sum(y1 * y1, axis=0, keepdims=True)
    st_ref[...] += jnp.concatenate([s1, s2], axis=0)


def _bn_conv2_kernel(y1_ref, st_ref, g_ref, b_ref, W2_ref,
                     y2_ref, st2_ref, *, count):
    b = pl.program_id(0)
    t = pl.program_id(1)

    @pl.when(jnp.logical_and(b == 0, t == 0))
    def _init():
        st2_ref[...] = jnp.zeros_like(st2_ref)

    st = st_ref[...]
    inv_cnt = 1.0 / count
    mean = st[0:1, :] * inv_cnt
    var = st[1:2, :] * inv_cnt - mean * mean
    inv = jax.lax.rsqrt(var + 1e-3)
    z = (y1_ref[0] - mean) * inv * g_ref[...] + b_ref[...]
    z = jnp.where(z >= 0, z, 0.01 * z)
    y2 = jax.lax.dot_general(z, W2_ref[...], (((1,), (1,)), ((), ())),
                             precision=_PREC)
    y2_ref[0] = y2
    s1 = jnp.sum(y2, axis=0, keepdims=True)
    s2 = jnp.sum(y2 * y2, axis=0, keepdims=True)
    st2_ref[...] += jnp.concatenate([s1, s2], axis=0)


def _bn_out_kernel(y2_ref, st_ref, g_ref, b_ref, out_ref, *, count):
    st = st_ref[...]
    inv_cnt = 1.0 / count
    mean = st[0:1, :] * inv_cnt
    var = st[1:2, :] * inv_cnt - mean * mean
    inv = jax.lax.rsqrt(var + 1e-3)
    z = (y2_ref[0] - mean) * inv * g_ref[...] + b_ref[...]
    z = jnp.where(z >= 0, z, 0.01 * z)
    out_ref[0] = z.T


def kernel(xyz, skip, xyz_prev, feat_prev, W1, g1, b1, W2, g2, b2):
    B, _, N0 = xyz.shape
    N = xyz_prev.shape[2]
    Cprev = feat_prev.shape[1]
    Cskip = skip.shape[1]
    C1 = W1.shape[0]
    C2 = W2.shape[0]
    TQ = 1024
    nt = N0 // TQ
    TC = 1024
    ntc = N0 // TC
    Q = B * N0
    count = float(Q)

    p_t = jnp.transpose(xyz_prev, (0, 2, 1))

    table = jnp.transpose(feat_prev, (0, 2, 1)).reshape(B * N, Cprev)

    idx_flat, w_flat = pl.pallas_call(
        functools.partial(_three_nn_kernel, n_keys=N, n_tiles=nt),
        grid=(B * nt,),
        in_specs=[pl.BlockSpec((1, 3, TQ), lambda i: (i // nt, 0, i % nt)),
                  pl.BlockSpec((1, N, 3), lambda i: (i // nt, 0, 0))],
        out_specs=[pl.BlockSpec((3, TQ), lambda i: (0, i)),
                   pl.BlockSpec((3, TQ), lambda i: (0, i))],
        out_shape=[jax.ShapeDtypeStruct((3, Q), jnp.int32),
                   jax.ShapeDtypeStruct((3, Q), jnp.float32)],
    )(xyz, p_t)

    feats = _sc_interp(table, idx_flat, w_flat, q_total=Q, d_feat=Cprev)
    feats = feats.reshape(B, N0, Cprev)

    y1, st1 = pl.pallas_call(
        functools.partial(_conv1_kernel, cprev=Cprev),
        grid=(B, ntc),
        in_specs=[pl.BlockSpec((1, TC, Cprev), lambda b, t: (b, t, 0)),
                  pl.BlockSpec((1, Cskip, TC), lambda b, t: (b, 0, t)),
                  pl.BlockSpec((C1, Cprev + Cskip), lambda b, t: (0, 0))],
        out_specs=[pl.BlockSpec((1, TC, C1), lambda b, t: (b, t, 0)),
                   pl.BlockSpec((2, C1), lambda b, t: (0, 0))],
        out_shape=[jax.ShapeDtypeStruct((B, N0, C1), jnp.float32),
                   jax.ShapeDtypeStruct((2, C1), jnp.float32)],
    )(feats, skip, W1)

    y2, st2 = pl.pallas_call(
        functools.partial(_bn_conv2_kernel, count=count),
        grid=(B, ntc),
        in_specs=[pl.BlockSpec((1, TC, C1), lambda b, t: (b, t, 0)),
                  pl.BlockSpec((2, C1), lambda b, t: (0, 0)),
                  pl.BlockSpec((1, C1), lambda b, t: (0, 0)),
                  pl.BlockSpec((1, C1), lambda b, t: (0, 0)),
                  pl.BlockSpec((C2, C1), lambda b, t: (0, 0))],
        out_specs=[pl.BlockSpec((1, TC, C2), lambda b, t: (b, t, 0)),
                   pl.BlockSpec((2, C2), lambda b, t: (0, 0))],
        out_shape=[jax.ShapeDtypeStruct((B, N0, C2), jnp.float32),
                   jax.ShapeDtypeStruct((2, C2), jnp.float32)],
    )(y1, st1, g1.reshape(1, -1), b1.reshape(1, -1), W2)

    y = pl.pallas_call(
        functools.partial(_bn_out_kernel, count=count),
        grid=(B, ntc),
        in_specs=[pl.BlockSpec((1, TC, C2), lambda b, t: (b, t, 0)),
                  pl.BlockSpec((2, C2), lambda b, t: (0, 0)),
                  pl.BlockSpec((1, C2), lambda b, t: (0, 0)),
                  pl.BlockSpec((1, C2), lambda b, t: (0, 0))],
        out_specs=pl.BlockSpec((1, C2, TC), lambda b, t: (b, 0, t)),
        out_shape=jax.ShapeDtypeStruct((B, C2, N0), jnp.float32),
    )(y2, st2, g2.reshape(1, -1), b2.reshape(1, -1))

    return (xyz, y)
